# baseline (speedup 1.0000x reference)
import functools

import jax
import jax.numpy as jnp
import numpy as np
from jax import lax
from jax.experimental import pallas as pl
from jax.experimental.pallas import tpu as pltpu
from jax.experimental.pallas import tpu_sc as plsc

VOCAB = 1000000
CHAR_DIM = 32
LATENT = 32
HIDDEN = 128
BATCH = 16384

_PACK = 128 // CHAR_DIM
_WIDE_ROWS = VOCAB // _PACK

_NC = 2
_NS = 16
_NW = _NC * _NS
_B_PER_W = BATCH // _NW
_CHUNK = 128
_NCHUNK = _B_PER_W // _CHUNK
_LANES = 16

_TC_BLK = 2048

with jax.default_device(jax.local_devices(backend="cpu")[0]):
    _EPS = np.asarray(
        jax.random.normal(jax.random.key(42), (BATCH, LATENT), dtype=jnp.float32))


def _sc_gather_wide(table_wide, indices):
    mesh = plsc.VectorSubcoreMesh(
        core_axis_name="c", subcore_axis_name="s",
        num_cores=_NC, num_subcores=_NS,
    )

    @functools.partial(
        pl.kernel,
        out_type=jax.ShapeDtypeStruct((BATCH, 128), jnp.float32),
        mesh=mesh,
        scratch_types=[
            pltpu.VMEM((_B_PER_W,), jnp.int32),
            pltpu.VMEM((_NCHUNK, _CHUNK), jnp.int32),
            pltpu.VMEM((_B_PER_W, 128), jnp.float32),
            pltpu.SemaphoreType.DMA,
        ],
    )
    def gather_kernel(table_hbm, idx_hbm, out_hbm, idx_v, widx_v, rows_v, sem):
        wid = lax.axis_index("s") * _NC + lax.axis_index("c")
        base = wid * _B_PER_W
        pltpu.sync_copy(idx_hbm.at[pl.ds(base, _B_PER_W)], idx_v)
        for m in range(_B_PER_W // _LANES):
            vec = idx_v[pl.ds(m * _LANES, _LANES)]
            j, o = divmod(m * _LANES, _CHUNK)
            widx_v[j, pl.ds(o, _LANES)] = lax.shift_right_logical(vec, 2)
        copies = [
            pltpu.async_copy(
                table_hbm.at[widx_v.at[j]],
                rows_v.at[pl.ds(j * _CHUNK, _CHUNK)],
                sem,
            )
            for j in range(_NCHUNK)
        ]
        for cp in copies:
            cp.wait()
        pltpu.sync_copy(rows_v, out_hbm.at[pl.ds(base, _B_PER_W)])

    return gather_kernel(table_wide, indices)


def _vae_body(xw_ref, idx_ref, eps_ref, encW, encb, muW, mub, varW, varb,
              dinW, dinb, decW, decb, finW, finb,
              x_ref, rec_ref, mu_ref, lv_ref):
    def leaky(a):
        return jnp.where(a > 0, a, 0.01 * a)

    xw = xw_ref[...]
    rem = jnp.bitwise_and(idx_ref[...], _PACK - 1)
    x = jnp.zeros((xw.shape[0], CHAR_DIM), jnp.float32)
    for k in range(_PACK):
        x = jnp.where(rem == k, xw[:, k * CHAR_DIM:(k + 1) * CHAR_DIM], x)
    h = jnp.dot(x, encW[...], preferred_element_type=jnp.float32) + encb[...]
    h = leaky(h)
    mu = jnp.dot(h, muW[...], preferred_element_type=jnp.float32) + mub[...]
    lv = jnp.dot(h, varW[...], preferred_element_type=jnp.float32) + varb[...]
    z = eps_ref[...] * jnp.exp(0.5 * lv) + mu
    d = jnp.dot(z, dinW[...], preferred_element_type=jnp.float32) + dinb[...]
    d = leaky(d)
    d = jnp.dot(d, decW[...], preferred_element_type=jnp.float32) + decb[...]
    d = leaky(d)
    rec_ref[...] = jnp.dot(d, finW[...], preferred_element_type=jnp.float32) + finb[...]
    x_ref[...] = x
    mu_ref[...] = mu
    lv_ref[...] = lv


def _vae_chain(xw, idx2d, enc_W, enc_b, mu_W, mu_b, var_W, var_b,
               din_W, din_b, dec_W, dec_b, fin_W, fin_b):
    grid = (BATCH // _TC_BLK,)
    blk = lambda c: pl.BlockSpec((_TC_BLK, c), lambda i: (i, 0))
    full = lambda a: pl.BlockSpec(a.shape, lambda i: (0,) * a.ndim)
    eps = jnp.asarray(_EPS)
    weights = (enc_W, enc_b.reshape(1, HIDDEN), mu_W, mu_b.reshape(1, LATENT),
               var_W, var_b.reshape(1, LATENT), din_W, din_b.reshape(1, HIDDEN),
               dec_W, dec_b.reshape(1, LATENT), fin_W, fin_b.reshape(1, CHAR_DIM))
    return pl.pallas_call(
        _vae_body,
        grid=grid,
        in_specs=[blk(128), blk(1), blk(LATENT)] + [full(w) for w in weights],
        out_specs=[blk(CHAR_DIM), blk(CHAR_DIM), blk(LATENT), blk(LATENT)],
        out_shape=[
            jax.ShapeDtypeStruct((BATCH, CHAR_DIM), jnp.float32),
            jax.ShapeDtypeStruct((BATCH, CHAR_DIM), jnp.float32),
            jax.ShapeDtypeStruct((BATCH, LATENT), jnp.float32),
            jax.ShapeDtypeStruct((BATCH, LATENT), jnp.float32),
        ],
    )(xw, idx2d, eps, *weights)


def kernel(indices, table, enc_W, enc_b, mu_W, mu_b, var_W, var_b,
           din_W, din_b, dec_W, dec_b, fin_W, fin_b):
    table_wide = table.reshape(_WIDE_ROWS, 128)
    xw = _sc_gather_wide(table_wide, indices)
    idx2d = indices.reshape(BATCH, 1)
    x, recons, mu, log_var = _vae_chain(
        xw, idx2d, enc_W, enc_b, mu_W, mu_b, var_W, var_b,
        din_W, din_b, dec_W, dec_b, fin_W, fin_b)
    return (recons, x, mu, log_var)
